# baseline (speedup 1.0000x reference)
import jax
import jax.numpy as jnp
from jax.experimental import pallas as pl
from jax.experimental.pallas import tpu as pltpu

_T, _D, _E = 32768, 768, 8
_BT = 2048


def _router_body(x_ref, w_ref, wout_ref, iout_ref):
    x = x_ref[...]
    w = w_ref[...]
    logits = jax.lax.dot_general(
        x, w, (((1,), (0,)), ((), ())), preferred_element_type=jnp.float32
    )
    col = jax.lax.broadcasted_iota(jnp.int32, logits.shape, 1)
    m1 = jnp.max(logits, axis=1, keepdims=True)
    i1 = jnp.min(jnp.where(logits == m1, col, _E), axis=1, keepdims=True)
    neg = jnp.full_like(logits, -jnp.inf)
    rest = jnp.where(col == i1, neg, logits)
    m2 = jnp.max(rest, axis=1, keepdims=True)
    i2 = jnp.min(jnp.where(rest == m2, col, _E), axis=1, keepdims=True)
    e = jnp.exp(m2 - m1)
    w1 = 1.0 / (1.0 + e)
    c2 = jax.lax.broadcasted_iota(jnp.int32, (wout_ref.shape[0], 2), 1)
    wout_ref[...] = jnp.where(c2 == 0, w1, 1.0 - w1)
    iout_ref[...] = jnp.where(c2 == 0, i1, i2)


def kernel(x_TD, kernel_DE):
    x = jnp.asarray(x_TD, jnp.float32)
    w = jnp.asarray(kernel_DE, jnp.float32)
    weights, experts = pl.pallas_call(
        _router_body,
        grid=(_T // _BT,),
        in_specs=[
            pl.BlockSpec((_BT, _D), lambda i: (i, 0)),
            pl.BlockSpec((_D, _E), lambda i: (0, 0)),
        ],
        out_specs=[
            pl.BlockSpec((_BT, 2), lambda i: (i, 0)),
            pl.BlockSpec((_BT, 2), lambda i: (i, 0)),
        ],
        out_shape=[
            jax.ShapeDtypeStruct((_T, 2), jnp.float32),
            jax.ShapeDtypeStruct((_T, 2), jnp.int32),
        ],
        compiler_params=pltpu.CompilerParams(
            dimension_semantics=("arbitrary",)
        ),
    )(x, w)
    return (weights, experts)


# speedup vs baseline: 1.0852x; 1.0852x over previous
import jax
import jax.numpy as jnp
from jax.experimental import pallas as pl
from jax.experimental.pallas import tpu as pltpu

_T, _D, _E = 32768, 768, 8
_BT = 4096


def _router_body(x_ref, w_ref, wout_ref, iout_ref):
    x = x_ref[...]
    w = w_ref[...]
    logits = jax.lax.dot_general(
        x, w, (((1,), (0,)), ((), ())), preferred_element_type=jnp.float32
    )
    col = jax.lax.broadcasted_iota(jnp.int32, logits.shape, 1)
    m1 = jnp.max(logits, axis=1, keepdims=True)
    i1 = jnp.min(jnp.where(logits == m1, col, _E), axis=1, keepdims=True)
    neg = jnp.full_like(logits, -jnp.inf)
    rest = jnp.where(col == i1, neg, logits)
    m2 = jnp.max(rest, axis=1, keepdims=True)
    i2 = jnp.min(jnp.where(rest == m2, col, _E), axis=1, keepdims=True)
    e = jnp.exp(m2 - m1)
    w1 = 1.0 / (1.0 + e)
    c2 = jax.lax.broadcasted_iota(jnp.int32, (wout_ref.shape[0], 2), 1)
    wout_ref[...] = jnp.where(c2 == 0, w1, 1.0 - w1)
    iout_ref[...] = jnp.where(c2 == 0, i1, i2)


def kernel(x_TD, kernel_DE):
    x = jnp.asarray(x_TD, jnp.float32)
    w = jnp.asarray(kernel_DE, jnp.float32)
    weights, experts = pl.pallas_call(
        _router_body,
        grid=(_T // _BT,),
        in_specs=[
            pl.BlockSpec((_BT, _D), lambda i: (i, 0)),
            pl.BlockSpec((_D, _E), lambda i: (0, 0)),
        ],
        out_specs=[
            pl.BlockSpec((_BT, 2), lambda i: (i, 0)),
            pl.BlockSpec((_BT, 2), lambda i: (i, 0)),
        ],
        out_shape=[
            jax.ShapeDtypeStruct((_T, 2), jnp.float32),
            jax.ShapeDtypeStruct((_T, 2), jnp.int32),
        ],
        compiler_params=pltpu.CompilerParams(
            dimension_semantics=("arbitrary",)
        ),
    )(x, w)
    return (weights, experts)
